# baseline (speedup 1.0000x reference)
import functools

import jax
import jax.numpy as jnp
from jax import lax
from jax.experimental import pallas as pl
from jax.experimental.pallas import tpu as pltpu
from jax.experimental.pallas import tpu_sc as plsc

_N_LEVELS = 255.0


_NW = 32
_LANES = 16
_CHUNK = 32768


def _sc_minmax_body(x_hbm, out_hbm, buf0, buf1, acc, sem0, sem1, *, n_chunks):
    wid = lax.axis_index("s") * 2 + lax.axis_index("c")
    base = wid * (n_chunks * _CHUNK)

    bufs = (buf0, buf1)
    sems = (sem0, sem1)
    copies = [None, None]

    def start(g):
        copies[g % 2] = pltpu.async_copy(
            x_hbm.at[pl.ds(base + g * _CHUNK, _CHUNK)], bufs[g % 2], sems[g % 2]
        )

    start(0)
    vmin = jnp.full((_LANES,), jnp.inf, jnp.float32)
    vmax = jnp.full((_LANES,), -jnp.inf, jnp.float32)
    for g in range(n_chunks):
        if g + 1 < n_chunks:
            start(g + 1)
        copies[g % 2].wait()
        buf = bufs[g % 2]

        def body(i, carry):
            mn, mx = carry
            v = buf[pl.ds(i * _LANES, _LANES)]
            return jnp.minimum(mn, v), jnp.maximum(mx, v)

        vmin, vmax = lax.fori_loop(
            0, _CHUNK // _LANES, body, (vmin, vmax), unroll=8
        )
    acc[0, :] = vmin
    acc[1, :] = vmax
    pltpu.sync_copy(acc, out_hbm.at[wid])


def _sc_minmax(x_flat):
    n = x_flat.shape[0]
    assert n % (_NW * _CHUNK) == 0
    n_chunks = n // (_NW * _CHUNK)
    mesh = plsc.VectorSubcoreMesh(core_axis_name="c", subcore_axis_name="s")
    kfn = functools.partial(
        pl.kernel,
        mesh=mesh,
        out_type=jax.ShapeDtypeStruct((_NW, 2, _LANES), jnp.float32),
        scratch_types=[
            pltpu.VMEM((_CHUNK,), jnp.float32),
            pltpu.VMEM((_CHUNK,), jnp.float32),
            pltpu.VMEM((2, _LANES), jnp.float32),
            pltpu.SemaphoreType.DMA,
            pltpu.SemaphoreType.DMA,
        ],
    )(functools.partial(_sc_minmax_body, n_chunks=n_chunks))
    return kfn(x_flat)


def _tc_minmax_body(x_ref, mm_ref, acc_ref, *, nb):
    i = pl.program_id(0)
    x = x_ref[...]
    bmn = jnp.min(x)
    bmx = jnp.max(x)

    @pl.when(i == 0)
    def _init():
        acc_ref[0] = bmn
        acc_ref[1] = bmx

    @pl.when(i > 0)
    def _acc():
        acc_ref[0] = jnp.minimum(acc_ref[0], bmn)
        acc_ref[1] = jnp.maximum(acc_ref[1], bmx)

    @pl.when(i == nb - 1)
    def _fin():
        mm_ref[0, 0] = acc_ref[0]
        mm_ref[0, 1] = acc_ref[1]


def _quant_body(so_ref, x_ref, o_ref):
    scale = so_ref[0, 0]
    offset = so_ref[0, 1]
    inv = 1.0 / scale
    x = x_ref[...]
    xi = jnp.round(x * inv) + offset
    xi = jnp.clip(xi, 0.0, _N_LEVELS)
    o_ref[...] = (xi - offset) * scale


_SC_ROWS = 1024


def kernel(x_f):
    rows, cols = x_f.shape
    tc_rows = rows - _SC_ROWS
    blk = 256

    sc_part = _sc_minmax(x_f[tc_rows:].reshape(-1))

    nb1 = tc_rows // blk
    tc_part = pl.pallas_call(
        functools.partial(_tc_minmax_body, nb=nb1),
        grid=(nb1,),
        in_specs=[pl.BlockSpec((blk, cols), lambda i: (i, 0))],
        out_specs=pl.BlockSpec(memory_space=pltpu.SMEM),
        out_shape=jax.ShapeDtypeStruct((1, 2), jnp.float32),
        scratch_shapes=[pltpu.SMEM((2,), jnp.float32)],
    )(x_f[:tc_rows])

    mn = jnp.minimum(tc_part[0, 0], jnp.min(sc_part[:, 0, :]))
    mx = jnp.maximum(tc_part[0, 1], jnp.max(sc_part[:, 1, :]))
    scale = (mx - mn) / _N_LEVELS
    offset = jnp.round(-mn / scale)
    so = jnp.stack([scale, offset]).reshape(1, 2)

    nb = rows // blk
    x_q = pl.pallas_call(
        _quant_body,
        grid=(nb,),
        in_specs=[
            pl.BlockSpec(memory_space=pltpu.SMEM),
            pl.BlockSpec((blk, cols), lambda i: (i, 0)),
        ],
        out_specs=pl.BlockSpec((blk, cols), lambda i: (i, 0)),
        out_shape=jax.ShapeDtypeStruct((rows, cols), jnp.float32),
    )(so, x_f)
    return x_q


# speedup vs baseline: 1.7118x; 1.7118x over previous
import functools

import jax
import jax.numpy as jnp
from jax import lax
from jax.experimental import pallas as pl
from jax.experimental.pallas import tpu as pltpu
from jax.experimental.pallas import tpu_sc as plsc

_N_LEVELS = 255.0


_NW = 32
_LANES = 16
_CHUNK_ROWS = 4


def _sc_minmax_body(x_hbm, out_hbm, buf0, buf1, acc, sem0, sem1, *,
                    row0, rows_per_worker, cols):
    wid = lax.axis_index("s") * 2 + lax.axis_index("c")
    base = row0 + wid * rows_per_worker
    n_chunks = rows_per_worker // _CHUNK_ROWS

    bufs = (buf0, buf1)
    sems = (sem0, sem1)
    copies = [None, None]

    def start(g):
        copies[g % 2] = pltpu.async_copy(
            x_hbm.at[pl.ds(base + g * _CHUNK_ROWS, _CHUNK_ROWS), :],
            bufs[g % 2],
            sems[g % 2],
        )

    start(0)
    vmin = jnp.full((_LANES,), jnp.inf, jnp.float32)
    vmax = jnp.full((_LANES,), -jnp.inf, jnp.float32)
    for g in range(n_chunks):
        if g + 1 < n_chunks:
            start(g + 1)
        copies[g % 2].wait()
        buf = bufs[g % 2]

        for r in range(_CHUNK_ROWS):
            def body(i, carry, _r=r):
                mn, mx = carry
                v = buf[_r, pl.ds(i * _LANES, _LANES)]
                return jnp.minimum(mn, v), jnp.maximum(mx, v)

            vmin, vmax = lax.fori_loop(
                0, cols // _LANES, body, (vmin, vmax), unroll=8
            )
    acc[0, :] = vmin
    acc[1, :] = vmax
    pltpu.sync_copy(acc, out_hbm.at[wid])


def _sc_minmax(x_f, row0):
    rows, cols = x_f.shape
    sc_rows = rows - row0
    assert sc_rows % (_NW * _CHUNK_ROWS) == 0
    rows_per_worker = sc_rows // _NW
    mesh = plsc.VectorSubcoreMesh(core_axis_name="c", subcore_axis_name="s")
    kfn = functools.partial(
        pl.kernel,
        mesh=mesh,
        out_type=jax.ShapeDtypeStruct((_NW, 2, _LANES), jnp.float32),
        scratch_types=[
            pltpu.VMEM((_CHUNK_ROWS, cols), jnp.float32),
            pltpu.VMEM((_CHUNK_ROWS, cols), jnp.float32),
            pltpu.VMEM((2, _LANES), jnp.float32),
            pltpu.SemaphoreType.DMA,
            pltpu.SemaphoreType.DMA,
        ],
    )(functools.partial(
        _sc_minmax_body, row0=row0, rows_per_worker=rows_per_worker, cols=cols))
    return kfn(x_f)


def _tc_minmax_body(x_ref, mm_ref, acc_ref, *, nb):
    i = pl.program_id(0)
    x = x_ref[...]
    bmn = jnp.min(x)
    bmx = jnp.max(x)

    @pl.when(i == 0)
    def _init():
        acc_ref[0] = bmn
        acc_ref[1] = bmx

    @pl.when(i > 0)
    def _acc():
        acc_ref[0] = jnp.minimum(acc_ref[0], bmn)
        acc_ref[1] = jnp.maximum(acc_ref[1], bmx)

    @pl.when(i == nb - 1)
    def _fin():
        mm_ref[0, 0] = acc_ref[0]
        mm_ref[0, 1] = acc_ref[1]


def _quant_body(so_ref, x_ref, o_ref):
    scale = so_ref[0, 0]
    offset = so_ref[0, 1]
    inv = 1.0 / scale
    x = x_ref[...]
    xi = jnp.round(x * inv) + offset
    xi = jnp.clip(xi, 0.0, _N_LEVELS)
    o_ref[...] = (xi - offset) * scale


_SC_ROWS = 768


def kernel(x_f):
    rows, cols = x_f.shape
    tc_rows = rows - _SC_ROWS
    blk = 256

    sc_part = _sc_minmax(x_f, tc_rows)

    nb1 = tc_rows // blk
    tc_part = pl.pallas_call(
        functools.partial(_tc_minmax_body, nb=nb1),
        grid=(nb1,),
        in_specs=[pl.BlockSpec((blk, cols), lambda i: (i, 0))],
        out_specs=pl.BlockSpec(memory_space=pltpu.SMEM),
        out_shape=jax.ShapeDtypeStruct((1, 2), jnp.float32),
        scratch_shapes=[pltpu.SMEM((2,), jnp.float32)],
    )(x_f)

    mn = jnp.minimum(tc_part[0, 0], jnp.min(sc_part[:, 0, :]))
    mx = jnp.maximum(tc_part[0, 1], jnp.max(sc_part[:, 1, :]))
    scale = (mx - mn) / _N_LEVELS
    offset = jnp.round(-mn / scale)
    so = jnp.stack([scale, offset]).reshape(1, 2)

    nb = rows // blk
    x_q = pl.pallas_call(
        _quant_body,
        grid=(nb,),
        in_specs=[
            pl.BlockSpec(memory_space=pltpu.SMEM),
            pl.BlockSpec((blk, cols), lambda i: (i, 0)),
        ],
        out_specs=pl.BlockSpec((blk, cols), lambda i: (i, 0)),
        out_shape=jax.ShapeDtypeStruct((rows, cols), jnp.float32),
    )(so, x_f)
    return x_q


# speedup vs baseline: 2.1255x; 1.2417x over previous
import functools

import jax
import jax.numpy as jnp
from jax.experimental import pallas as pl
from jax.experimental.pallas import tpu as pltpu

_N_LEVELS = 255.0

_BLK = 128
_K = 9


def _fused_body(x_ref, o_ref, acc_ref, cache_ref, st0_ref, st1_ref,
                so_ref, sem0, sem1, *, nb, blk, cols, k):
    s = pl.program_id(0)
    first_cached = nb - 1 - k

    @pl.when(s < nb)
    def _phase1():
        x = x_ref[...]
        mn = jnp.min(x, axis=0, keepdims=True)
        mx = jnp.max(x, axis=0, keepdims=True)

        @pl.when(s == 0)
        def _():
            acc_ref[0:1, :] = mn
            acc_ref[1:2, :] = mx

        @pl.when(s > 0)
        def _():
            acc_ref[0:1, :] = jnp.minimum(acc_ref[0:1, :], mn)
            acc_ref[1:2, :] = jnp.maximum(acc_ref[1:2, :], mx)

        @pl.when((s >= first_cached) & (s < nb - 1))
        def _():
            c = s - first_cached
            cache_ref[pl.ds(c * blk, blk), :] = x

        @pl.when(s == nb - 1)
        def _():
            gmn = jnp.min(acc_ref[0, :])
            gmx = jnp.max(acc_ref[1, :])
            scale = (gmx - gmn) / _N_LEVELS
            so_ref[0] = scale
            so_ref[1] = jnp.round(-gmn / scale)

    @pl.when(s >= nb)
    def _phase2():
        t = s - nb
        scale = so_ref[0]
        offset = so_ref[1]
        inv = 1.0 / scale

        def qd(x):
            xi = jnp.round(x * inv) + offset
            xi = jnp.clip(xi, 0.0, _N_LEVELS)
            return (xi - offset) * scale

        jw = jnp.where(
            t == 0,
            nb - 1,
            jnp.where(t <= k, first_cached + t - 1, t - k - 1),
        )

        @pl.when((t >= 2) & (t % 2 == 0))
        def _():
            pltpu.make_async_copy(
                st0_ref, o_ref.at[pl.ds(0, blk), :], sem0).wait()

        @pl.when((t >= 2) & (t % 2 == 1))
        def _():
            pltpu.make_async_copy(
                st1_ref, o_ref.at[pl.ds(0, blk), :], sem1).wait()

        st = [st0_ref, st1_ref]

        @pl.when(t == 0)
        def _():
            st[0][...] = qd(x_ref[...])

        @pl.when((t > 0) & (t <= k))
        def _():
            c = t - 1
            x = cache_ref[pl.ds(c * blk, blk), :]
            @pl.when(t % 2 == 0)
            def _():
                st0_ref[...] = qd(x)
            @pl.when(t % 2 == 1)
            def _():
                st1_ref[...] = qd(x)

        @pl.when(t > k)
        def _():
            @pl.when(t % 2 == 0)
            def _():
                st0_ref[...] = qd(x_ref[...])
            @pl.when(t % 2 == 1)
            def _():
                st1_ref[...] = qd(x_ref[...])

        @pl.when(t % 2 == 0)
        def _():
            pltpu.make_async_copy(
                st0_ref, o_ref.at[pl.ds(jw * blk, blk), :], sem0).start()

        @pl.when(t % 2 == 1)
        def _():
            pltpu.make_async_copy(
                st1_ref, o_ref.at[pl.ds(jw * blk, blk), :], sem1).start()

        @pl.when(t == nb - 1)
        def _():
            pltpu.make_async_copy(
                st0_ref, o_ref.at[pl.ds(0, blk), :], sem0).wait()
            pltpu.make_async_copy(
                st1_ref, o_ref.at[pl.ds(0, blk), :], sem1).wait()


def kernel(x_f):
    rows, cols = x_f.shape
    blk = _BLK
    nb = rows // blk
    k = min(_K, nb - 2)

    def imap(s):
        j = jnp.where(s < nb, s,
                      jnp.where(s <= nb + k, nb - 1, s - (nb + k + 1)))
        return (j, 0)

    x_q = pl.pallas_call(
        functools.partial(_fused_body, nb=nb, blk=blk, cols=cols, k=k),
        grid=(2 * nb,),
        in_specs=[pl.BlockSpec((blk, cols), imap)],
        out_specs=pl.BlockSpec(memory_space=pl.ANY),
        out_shape=jax.ShapeDtypeStruct((rows, cols), jnp.float32),
        scratch_shapes=[
            pltpu.VMEM((2, cols), jnp.float32),
            pltpu.VMEM((max(k, 1) * blk, cols), jnp.float32),
            pltpu.VMEM((blk, cols), jnp.float32),
            pltpu.VMEM((blk, cols), jnp.float32),
            pltpu.SMEM((2,), jnp.float32),
            pltpu.SemaphoreType.DMA,
            pltpu.SemaphoreType.DMA,
        ],
        compiler_params=pltpu.CompilerParams(
            dimension_semantics=("arbitrary",),
        ),
    )(x_f)
    return x_q


# speedup vs baseline: 2.1499x; 1.0114x over previous
import functools

import jax
import jax.numpy as jnp
from jax.experimental import pallas as pl
from jax.experimental.pallas import tpu as pltpu

_N_LEVELS = 255.0

_BLK = 128
_K = 9


def _fused_body(x_ref, o_ref, acc_ref, cache_ref, st0_ref, st1_ref,
                so_ref, sem0, sem1, *, nb, blk, cols, k):
    s = pl.program_id(0)
    first_cached = nb - 1 - k

    @pl.when(s < nb)
    def _phase1():
        x = x_ref[...]
        mn = x
        mx = x
        r = blk
        while r > 8:
            h = r // 2
            mn = jnp.minimum(mn[0:h, :], mn[h:r, :])
            mx = jnp.maximum(mx[0:h, :], mx[h:r, :])
            r = h

        @pl.when(s == 0)
        def _():
            acc_ref[0:8, :] = mn
            acc_ref[8:16, :] = mx

        @pl.when(s > 0)
        def _():
            acc_ref[0:8, :] = jnp.minimum(acc_ref[0:8, :], mn)
            acc_ref[8:16, :] = jnp.maximum(acc_ref[8:16, :], mx)

        @pl.when((s >= first_cached) & (s < nb - 1))
        def _():
            c = s - first_cached
            cache_ref[pl.ds(c * blk, blk), :] = x

        @pl.when(s == nb - 1)
        def _():
            gmn = jnp.min(acc_ref[0:8, :])
            gmx = jnp.max(acc_ref[8:16, :])
            scale = (gmx - gmn) / _N_LEVELS
            so_ref[0] = scale
            so_ref[1] = jnp.round(-gmn / scale)

    @pl.when(s >= nb)
    def _phase2():
        t = s - nb
        scale = so_ref[0]
        offset = so_ref[1]
        inv = 1.0 / scale
        lo = (0.0 - offset) * scale
        hi = (_N_LEVELS - offset) * scale

        def qd(x):
            y = jnp.round(x * inv) * scale
            return jnp.minimum(jnp.maximum(y, lo), hi)

        jw = jnp.where(
            t == 0,
            nb - 1,
            jnp.where(t <= k, first_cached + t - 1, t - k - 1),
        )

        @pl.when((t >= 2) & (t % 2 == 0))
        def _():
            pltpu.make_async_copy(
                st0_ref, o_ref.at[pl.ds(0, blk), :], sem0).wait()

        @pl.when((t >= 2) & (t % 2 == 1))
        def _():
            pltpu.make_async_copy(
                st1_ref, o_ref.at[pl.ds(0, blk), :], sem1).wait()

        st = [st0_ref, st1_ref]

        @pl.when(t == 0)
        def _():
            st[0][...] = qd(x_ref[...])

        @pl.when((t > 0) & (t <= k))
        def _():
            c = t - 1
            x = cache_ref[pl.ds(c * blk, blk), :]
            @pl.when(t % 2 == 0)
            def _():
                st0_ref[...] = qd(x)
            @pl.when(t % 2 == 1)
            def _():
                st1_ref[...] = qd(x)

        @pl.when(t > k)
        def _():
            @pl.when(t % 2 == 0)
            def _():
                st0_ref[...] = qd(x_ref[...])
            @pl.when(t % 2 == 1)
            def _():
                st1_ref[...] = qd(x_ref[...])

        @pl.when(t % 2 == 0)
        def _():
            pltpu.make_async_copy(
                st0_ref, o_ref.at[pl.ds(jw * blk, blk), :], sem0).start()

        @pl.when(t % 2 == 1)
        def _():
            pltpu.make_async_copy(
                st1_ref, o_ref.at[pl.ds(jw * blk, blk), :], sem1).start()

        @pl.when(t == nb - 1)
        def _():
            pltpu.make_async_copy(
                st0_ref, o_ref.at[pl.ds(0, blk), :], sem0).wait()
            pltpu.make_async_copy(
                st1_ref, o_ref.at[pl.ds(0, blk), :], sem1).wait()


def kernel(x_f):
    rows, cols = x_f.shape
    blk = _BLK
    nb = rows // blk
    k = min(_K, nb - 2)

    def imap(s):
        j = jnp.where(s < nb, s,
                      jnp.where(s <= nb + k, nb - 1, s - (nb + k + 1)))
        return (j, 0)

    x_q = pl.pallas_call(
        functools.partial(_fused_body, nb=nb, blk=blk, cols=cols, k=k),
        grid=(2 * nb,),
        in_specs=[pl.BlockSpec((blk, cols), imap)],
        out_specs=pl.BlockSpec(memory_space=pl.ANY),
        out_shape=jax.ShapeDtypeStruct((rows, cols), jnp.float32),
        scratch_shapes=[
            pltpu.VMEM((16, cols), jnp.float32),
            pltpu.VMEM((max(k, 1) * blk, cols), jnp.float32),
            pltpu.VMEM((blk, cols), jnp.float32),
            pltpu.VMEM((blk, cols), jnp.float32),
            pltpu.SMEM((2,), jnp.float32),
            pltpu.SemaphoreType.DMA,
            pltpu.SemaphoreType.DMA,
        ],
        compiler_params=pltpu.CompilerParams(
            dimension_semantics=("arbitrary",),
        ),
    )(x_f)
    return x_q
